# baseline (speedup 1.0000x reference)
import functools

import jax
import jax.numpy as jnp
from jax import lax
from jax.experimental import pallas as pl
from jax.experimental.pallas import tpu as pltpu
from jax.experimental.pallas import tpu_sc as plsc

NEXP = 16
NF = 12
TOPK = 2
D = 1024
H = D // 2
F = 2048
T = 4096
TH = T // 2
B = 128
NBH = (TH * TOPK) // B + NF
NPADH = NBH * B
ZROWH = NPADH - 1
RB = 512
NW = 32
_HI = -65536

_SC_MESH = plsc.VectorSubcoreMesh(core_axis_name="c", subcore_axis_name="s")


def _pack_cols(v):
    r = v.astype(jnp.bfloat16).astype(jnp.float32)
    u = lax.bitcast_convert_type(r, jnp.uint32)
    return lax.bitcast_convert_type(
        (u[:, :H] >> 16) | u[:, H:], jnp.int32)


def _unpack_cols(p):
    u = lax.bitcast_convert_type(p, jnp.uint32)
    lo = lax.bitcast_convert_type(u << 16, jnp.float32)
    hi = lax.bitcast_convert_type(u & jnp.uint32(0xFFFF0000), jnp.float32)
    return jnp.concatenate([lo, hi], axis=1)


def _router_body(x_ref, wg_ref, cwf_ref, cconst_ref,
                 cheap_ref, gates_ref, idx_ref, xp_ref):
    xb = x_ref[...]
    logits = jnp.dot(xb, wg_ref[...], preferred_element_type=jnp.float32)
    m = jnp.max(logits, axis=1, keepdims=True)
    ex = jnp.exp(logits - m)
    p = ex / jnp.sum(ex, axis=1, keepdims=True)
    iota = lax.broadcasted_iota(jnp.int32, (RB, NEXP), 1)
    g1 = jnp.max(p, axis=1, keepdims=True)
    i1 = jnp.min(jnp.where(p == g1, iota, NEXP), axis=1, keepdims=True)
    p2 = jnp.where(iota == i1, -jnp.inf, p)
    g2 = jnp.max(p2, axis=1, keepdims=True)
    i2 = jnp.min(jnp.where(p2 == g2, iota, NEXP), axis=1, keepdims=True)
    g1z = jnp.where(i1 == NEXP - 1, 0.0, g1)
    g2z = jnp.where(i2 == NEXP - 1, 0.0, g2)
    s = g1z + g2z
    gn1 = g1z / s
    gn2 = g2z / s
    t2 = xb * 2.0
    cl = jnp.dot(t2, cwf_ref[...], preferred_element_type=jnp.float32)
    cc = cconst_ref[...]
    cheap = jnp.zeros_like(xb)
    for j in range(2):
        lj = cl[:, 2 * j:2 * j + 2]
        mj = jnp.max(lj, axis=1, keepdims=True)
        ej = jnp.exp(lj - mj)
        wj = ej / jnp.sum(ej, axis=1, keepdims=True)
        ge = (jnp.where(i1 == NF + j, gn1, 0.0)
              + jnp.where(i2 == NF + j, gn2, 0.0))
        cheap = cheap + ge * (wj[:, 0:1] * t2 + wj[:, 1:2] * cc[j:j + 1, :])
    ge_c = (jnp.where(i1 == NEXP - 2, gn1, 0.0)
            + jnp.where(i2 == NEXP - 2, gn2, 0.0))
    cheap = cheap + ge_c * t2
    cheap_ref[...] = cheap
    gates_ref[...] = jnp.concatenate([gn1, gn2], axis=1)
    idx_ref[...] = jnp.concatenate([i1, i2], axis=1).astype(jnp.int32)
    xp_ref[...] = _pack_cols(xb)


def _router(xf, wg, cwf, cconst):
    return pl.pallas_call(
        _router_body,
        grid=(T // RB,),
        in_specs=[
            pl.BlockSpec((RB, D), lambda i: (i, 0)),
            pl.BlockSpec((D, NEXP), lambda i: (0, 0)),
            pl.BlockSpec((D, 4), lambda i: (0, 0)),
            pl.BlockSpec((2, D), lambda i: (0, 0)),
        ],
        out_specs=[
            pl.BlockSpec((RB, D), lambda i: (i, 0)),
            pl.BlockSpec((RB, TOPK), lambda i: (i, 0)),
            pl.BlockSpec((RB, TOPK), lambda i: (i, 0)),
            pl.BlockSpec((RB, H), lambda i: (i, 0)),
        ],
        out_shape=[
            jax.ShapeDtypeStruct((T, D), jnp.float32),
            jax.ShapeDtypeStruct((T, TOPK), jnp.float32),
            jax.ShapeDtypeStruct((T, TOPK), jnp.int32),
            jax.ShapeDtypeStruct((T, H), jnp.int32),
        ],
    )(xf, wg, cwf, cconst)


GPWH = NPADH // NW
G_STAGES = ((0, 88), (88, 88))


@functools.partial(
    pl.kernel,
    mesh=_SC_MESH,
    out_type=jax.ShapeDtypeStruct((NPADH, H), jnp.int32),
    scratch_types=(
        [pltpu.VMEM((GPWH,), jnp.int32)]
        + [pltpu.VMEM((88, H), jnp.int32) for _ in range(2)]
        + [pltpu.SemaphoreType.DMA for _ in range(4)]
    ),
)
def _sc_gather(xp_hbm, tok_hbm, xs_hbm, idx_v, b0, b1, g0s, g1s, w0s, w1s):
    bufs = (b0, b1)
    gsems = (g0s, g1s)
    wsems = (w0s, w1s)
    wid = lax.axis_index("s") * 2 + lax.axis_index("c")
    base = wid * GPWH
    pltpu.sync_copy(tok_hbm.at[pl.ds(base, GPWH)], idx_v)
    gh = {}
    wh = {}

    def issue_gather(c):
        off, sz = G_STAGES[c]
        gh[c] = pltpu.async_copy(
            xp_hbm.at[idx_v.at[pl.ds(off, sz)]],
            bufs[c % 2].at[pl.ds(0, sz)], gsems[c % 2])

    issue_gather(0)
    issue_gather(1)
    for c in range(len(G_STAGES)):
        off, sz = G_STAGES[c]
        gh[c].wait()
        wh[c] = pltpu.async_copy(
            bufs[c % 2].at[pl.ds(0, sz)],
            xs_hbm.at[pl.ds(base + off, sz)], wsems[c % 2])
    wh[0].wait()
    wh[1].wait()


def _ffn_body(be_ref, na_ref, xs_ref, w1_ref, b1_ref, w2_ref, b2_ref, g_ref,
              ys_ref):
    b = pl.program_id(0)

    @pl.when(b < na_ref[0])
    def _compute():
        xb = (_unpack_cols(xs_ref[...]) * 2.0).astype(jnp.bfloat16)
        h = jnp.dot(xb, w1_ref[0], preferred_element_type=jnp.float32)
        h = jnp.maximum(h + b1_ref[0], 0.0).astype(jnp.bfloat16)
        y = jnp.dot(h, w2_ref[0], preferred_element_type=jnp.float32)
        ys_ref[...] = _pack_cols((y + b2_ref[0]) * g_ref[...])

    @pl.when(b >= na_ref[0])
    def _zero():
        ys_ref[...] = jnp.zeros_like(ys_ref)


def _ffn(block_expert, n_active, xs, W1b, b1r, W2b, b2r, gate_col):
    grid_spec = pltpu.PrefetchScalarGridSpec(
        num_scalar_prefetch=2,
        grid=(NBH,),
        in_specs=[
            pl.BlockSpec((B, H), lambda b, be, na: (b, 0)),
            pl.BlockSpec((1, D, F), lambda b, be, na: (be[b], 0, 0)),
            pl.BlockSpec((1, 1, F), lambda b, be, na: (be[b], 0, 0)),
            pl.BlockSpec((1, F, D), lambda b, be, na: (be[b], 0, 0)),
            pl.BlockSpec((1, 1, D), lambda b, be, na: (be[b], 0, 0)),
            pl.BlockSpec((B, 1), lambda b, be, na: (b, 0)),
        ],
        out_specs=pl.BlockSpec((B, H), lambda b, be, na: (b, 0)),
    )
    return pl.pallas_call(
        _ffn_body,
        grid_spec=grid_spec,
        out_shape=jax.ShapeDtypeStruct((NPADH, H), jnp.int32),
        compiler_params=pltpu.CompilerParams(
            dimension_semantics=("arbitrary",)),
    )(block_expert, n_active, xs, W1b, b1r, W2b, b2r, gate_col)


CC = 16
TPWH = TH // NW
CCHH = TPWH // CC
CNS = 4


@functools.partial(
    pl.kernel,
    mesh=_SC_MESH,
    out_type=jax.ShapeDtypeStruct((TH, H), jnp.int32),
    scratch_types=(
        [pltpu.VMEM((TPWH,), jnp.int32), pltpu.VMEM((TPWH,), jnp.int32)]
        + [pltpu.VMEM((CC, H), jnp.int32) for _ in range(2 * CNS)]
        + [pltpu.SemaphoreType.DMA for _ in range(2 * CNS)]
    ),
)
def _sc_pairsum(ys_hbm, pos0_hbm, pos1_hbm, out_hbm, p0_v, p1_v, *rest):
    r0s = rest[:CNS]
    r1s = rest[CNS:2 * CNS]
    dsems = rest[2 * CNS:3 * CNS]
    wsems = rest[3 * CNS:]
    wid = lax.axis_index("s") * 2 + lax.axis_index("c")
    base = wid * TPWH
    pltpu.sync_copy(pos0_hbm.at[pl.ds(base, TPWH)], p0_v)
    pltpu.sync_copy(pos1_hbm.at[pl.ds(base, TPWH)], p1_v)
    dh = {}
    wh = {}

    def issue(c):
        k = c % CNS
        dh[c] = (
            pltpu.async_copy(
                ys_hbm.at[p0_v.at[pl.ds(c * CC, CC)]], r0s[k], dsems[k]),
            pltpu.async_copy(
                ys_hbm.at[p1_v.at[pl.ds(c * CC, CC)]], r1s[k], dsems[k]),
        )

    for c in range(min(CNS, CCHH)):
        issue(c)
    for c in range(CCHH):
        if 1 <= c and c + CNS - 1 < CCHH:
            wh[c - 1].wait()
            issue(c + CNS - 1)
        for hnd in dh[c]:
            hnd.wait()
        k = c % CNS
        r0, r1 = r0s[k], r1s[k]

        def _rne16(u):
            return (u + 0x7FFF + ((u >> 16) & 1)) >> 16

        def _row(i, _):
            for j in range(H // 16):
                sl = pl.ds(j * 16, 16)
                v0 = r0[i, sl]
                v1 = r1[i, sl]
                lo = (lax.bitcast_convert_type(v0 << 16, jnp.float32)
                      + lax.bitcast_convert_type(v1 << 16, jnp.float32))
                hi = (lax.bitcast_convert_type(v0 & _HI, jnp.float32)
                      + lax.bitcast_convert_type(v1 & _HI, jnp.float32))
                ulo = lax.bitcast_convert_type(lo, jnp.uint32)
                uhi = lax.bitcast_convert_type(hi, jnp.uint32)
                packed = _rne16(ulo) | (_rne16(uhi) << 16)
                r0[i, sl] = lax.bitcast_convert_type(packed, jnp.int32)
            return 0

        lax.fori_loop(0, CC, _row, 0)
        wh[c] = pltpu.async_copy(
            r0, out_hbm.at[pl.ds(base + c * CC, CC)], wsems[k])
    for c in range(max(0, CCHH - CNS), CCHH):
        wh[c].wait()


def _finish_body(cheap_ref, rs_ref, out_ref):
    out_ref[...] = cheap_ref[...] + _unpack_cols(rs_ref[...])


def _make_finish(base):
    return pl.pallas_call(
        _finish_body,
        grid=(TH // RB,),
        in_specs=[
            pl.BlockSpec((RB, D), lambda i: (base + i, 0)),
            pl.BlockSpec((RB, H), lambda i: (i, 0)),
        ],
        out_specs=pl.BlockSpec((RB, D), lambda i: (i, 0)),
        out_shape=jax.ShapeDtypeStruct((TH, D), jnp.float32),
    )


def _route_tables(idx_h, gates_h, half):
    pair_e = idx_h.reshape(-1)
    pair_g = gates_h.reshape(-1)
    pair_t = jnp.repeat(
        jnp.arange(TH, dtype=jnp.int32) + jnp.int32(half * TH), TOPK)
    is_ffn = pair_e < NF
    ec = jnp.where(is_ffn, pair_e, 0)
    onehot = (pair_e[:, None]
              == jnp.arange(NF, dtype=jnp.int32)[None, :]).astype(jnp.int32)
    csum = jnp.cumsum(onehot, axis=0)
    rank = jnp.take_along_axis(csum, ec[:, None], axis=1)[:, 0] - 1
    counts = csum[-1]
    padded = ((counts + B - 1) // B) * B
    po = jnp.concatenate(
        [jnp.zeros((1,), jnp.int32), jnp.cumsum(padded)]).astype(jnp.int32)
    dest = po[ec] + rank
    dest_s = jnp.where(is_ffn, dest, NPADH)
    tok_sorted = jnp.zeros((NPADH,), jnp.int32).at[dest_s].set(
        pair_t, mode="drop")
    gate_sorted = jnp.zeros((NPADH,), jnp.float32).at[dest_s].set(
        pair_g, mode="drop")
    pos = jnp.where(is_ffn, dest, ZROWH).reshape(TH, TOPK)
    n_active = (po[NF] // B).reshape(1).astype(jnp.int32)
    bstart = jnp.arange(NBH, dtype=jnp.int32) * B
    block_expert = jnp.minimum(
        jnp.sum((bstart[:, None] >= po[None, 1:NF + 1]).astype(jnp.int32),
                axis=1),
        NF - 1).astype(jnp.int32)
    return (tok_sorted, gate_sorted, pos[:, 0] + 0, pos[:, 1] + 0,
            n_active, block_expert)


def kernel(x, wg, W1, b1, W2, b2, cw, cconst):
    xf = x.reshape(T, D)
    cwf = jnp.concatenate([cw[0], cw[1]], axis=1)
    cheap, gates, idx, xp = _router(xf, wg, cwf, cconst)

    W1b = W1.astype(jnp.bfloat16)
    W2b = W2.astype(jnp.bfloat16)
    b1r = b1.reshape(NF, 1, F)
    b2r = b2.reshape(NF, 1, D)

    outs = []
    for half in range(2):
        sl = slice(half * TH, (half + 1) * TH)
        tok_s, gate_s, pos0, pos1, n_act, blk_e = _route_tables(
            idx[sl], gates[sl], half)
        xs = _sc_gather(xp, tok_s)
        ys = _ffn(blk_e, n_act, xs, W1b, b1r, W2b, b2r, gate_s[:, None])
        rsum = _sc_pairsum(ys, pos0, pos1)
        outs.append(_make_finish(half * (TH // RB))(cheap, rsum))
    return jnp.stack(outs).reshape(x.shape)


# speedup vs baseline: 1.3588x; 1.3588x over previous
import functools

import jax
import jax.numpy as jnp
from jax import lax
from jax.experimental import pallas as pl
from jax.experimental.pallas import tpu as pltpu
from jax.experimental.pallas import tpu_sc as plsc

NEXP = 16
NF = 12
TOPK = 2
D = 1024
H = D // 2
F = 2048
T = 4096
B = 128
NB = (T * TOPK) // B + NF
NPAD = NB * B
ZROW = NPAD - 1
RB = 512
NW = 32

_SC_MESH = plsc.VectorSubcoreMesh(core_axis_name="c", subcore_axis_name="s")


def _pack_cols(v):
    r = v.astype(jnp.bfloat16).astype(jnp.float32)
    u = lax.bitcast_convert_type(r, jnp.uint32)
    return lax.bitcast_convert_type(
        (u[:, :H] >> 16) | u[:, H:], jnp.int32)


def _unpack_cols(p):
    u = lax.bitcast_convert_type(p, jnp.uint32)
    lo = lax.bitcast_convert_type(u << 16, jnp.float32)
    hi = lax.bitcast_convert_type(u & jnp.uint32(0xFFFF0000), jnp.float32)
    return jnp.concatenate([lo, hi], axis=1)


def _router_body(x_ref, wg_ref, cwf_ref, cconst_ref,
                 cheap_ref, gates_ref, idx_ref, xp_ref):
    xb = x_ref[...]
    logits = jnp.dot(xb, wg_ref[...], preferred_element_type=jnp.float32)
    m = jnp.max(logits, axis=1, keepdims=True)
    ex = jnp.exp(logits - m)
    p = ex / jnp.sum(ex, axis=1, keepdims=True)
    iota = lax.broadcasted_iota(jnp.int32, (RB, NEXP), 1)
    g1 = jnp.max(p, axis=1, keepdims=True)
    i1 = jnp.min(jnp.where(p == g1, iota, NEXP), axis=1, keepdims=True)
    p2 = jnp.where(iota == i1, -jnp.inf, p)
    g2 = jnp.max(p2, axis=1, keepdims=True)
    i2 = jnp.min(jnp.where(p2 == g2, iota, NEXP), axis=1, keepdims=True)
    g1z = jnp.where(i1 == NEXP - 1, 0.0, g1)
    g2z = jnp.where(i2 == NEXP - 1, 0.0, g2)
    s = g1z + g2z
    gn1 = g1z / s
    gn2 = g2z / s
    t2 = xb * 2.0
    cl = jnp.dot(t2, cwf_ref[...], preferred_element_type=jnp.float32)
    cc = cconst_ref[...]
    cheap = jnp.zeros_like(xb)
    for j in range(2):
        lj = cl[:, 2 * j:2 * j + 2]
        mj = jnp.max(lj, axis=1, keepdims=True)
        ej = jnp.exp(lj - mj)
        wj = ej / jnp.sum(ej, axis=1, keepdims=True)
        ge = (jnp.where(i1 == NF + j, gn1, 0.0)
              + jnp.where(i2 == NF + j, gn2, 0.0))
        cheap = cheap + ge * (wj[:, 0:1] * t2 + wj[:, 1:2] * cc[j:j + 1, :])
    ge_c = (jnp.where(i1 == NEXP - 2, gn1, 0.0)
            + jnp.where(i2 == NEXP - 2, gn2, 0.0))
    cheap = cheap + ge_c * t2
    cheap_ref[...] = cheap
    gates_ref[...] = jnp.concatenate([gn1, gn2], axis=1)
    idx_ref[...] = jnp.concatenate([i1, i2], axis=1).astype(jnp.int32)
    xp_ref[...] = _pack_cols(xb)


def _router(xf, wg, cwf, cconst):
    return pl.pallas_call(
        _router_body,
        grid=(T // RB,),
        in_specs=[
            pl.BlockSpec((RB, D), lambda i: (i, 0)),
            pl.BlockSpec((D, NEXP), lambda i: (0, 0)),
            pl.BlockSpec((D, 4), lambda i: (0, 0)),
            pl.BlockSpec((2, D), lambda i: (0, 0)),
        ],
        out_specs=[
            pl.BlockSpec((RB, D), lambda i: (i, 0)),
            pl.BlockSpec((RB, TOPK), lambda i: (i, 0)),
            pl.BlockSpec((RB, TOPK), lambda i: (i, 0)),
            pl.BlockSpec((RB, H), lambda i: (i, 0)),
        ],
        out_shape=[
            jax.ShapeDtypeStruct((T, D), jnp.float32),
            jax.ShapeDtypeStruct((T, TOPK), jnp.float32),
            jax.ShapeDtypeStruct((T, TOPK), jnp.int32),
            jax.ShapeDtypeStruct((T, H), jnp.int32),
        ],
    )(xf, wg, cwf, cconst)


GPW = NPAD // NW
G_STAGES = ((0, 104), (104, 104), (208, 96))
GMAX = 104


@functools.partial(
    pl.kernel,
    mesh=_SC_MESH,
    out_type=jax.ShapeDtypeStruct((NPAD, H), jnp.int32),
    scratch_types=(
        [pltpu.VMEM((GPW,), jnp.int32)]
        + [pltpu.VMEM((GMAX, H), jnp.int32) for _ in range(2)]
        + [pltpu.SemaphoreType.DMA for _ in range(4)]
    ),
)
def _sc_gather(xp_hbm, tok_hbm, xs_hbm, idx_v, b0, b1, g0s, g1s, w0s, w1s):
    bufs = (b0, b1)
    gsems = (g0s, g1s)
    wsems = (w0s, w1s)
    wid = lax.axis_index("s") * 2 + lax.axis_index("c")
    base = wid * GPW
    pltpu.sync_copy(tok_hbm.at[pl.ds(base, GPW)], idx_v)
    gh = {}
    wh = {}

    def issue_gather(c):
        off, sz = G_STAGES[c]
        gh[c] = pltpu.async_copy(
            xp_hbm.at[idx_v.at[pl.ds(off, sz)]],
            bufs[c % 2].at[pl.ds(0, sz)], gsems[c % 2])

    issue_gather(0)
    issue_gather(1)
    for c in range(len(G_STAGES)):
        off, sz = G_STAGES[c]
        gh[c].wait()
        wh[c] = pltpu.async_copy(
            bufs[c % 2].at[pl.ds(0, sz)],
            xs_hbm.at[pl.ds(base + off, sz)], wsems[c % 2])
        if c + 2 < len(G_STAGES):
            wh[c].wait()
            issue_gather(c + 2)
    wh[len(G_STAGES) - 2].wait()
    wh[len(G_STAGES) - 1].wait()


def _ffn_body(be_ref, na_ref, xs_ref, w1_ref, b1_ref, w2_ref, b2_ref, g_ref,
              ys_ref, w1c_ref, w2c_ref):
    b = pl.program_id(0)

    @pl.when(b < na_ref[0])
    def _compute():
        changed = jnp.logical_or(
            b == 0, be_ref[b] != be_ref[jnp.maximum(b - 1, 0)])

        @pl.when(changed)
        def _cast():
            w1c_ref[...] = w1_ref[0].astype(jnp.bfloat16)
            w2c_ref[...] = w2_ref[0].astype(jnp.bfloat16)

        xb = (_unpack_cols(xs_ref[...]) * 2.0).astype(jnp.bfloat16)
        h = jnp.dot(xb, w1c_ref[...], preferred_element_type=jnp.float32)
        h = jnp.maximum(h + b1_ref[0], 0.0).astype(jnp.bfloat16)
        y = jnp.dot(h, w2c_ref[...], preferred_element_type=jnp.float32)
        ys_ref[...] = _pack_cols((y + b2_ref[0]) * g_ref[...])

    @pl.when(b >= na_ref[0])
    def _zero():
        ys_ref[...] = jnp.zeros_like(ys_ref)


def _ffn(block_expert, n_active, xs, W1, b1r, W2, b2r, gate_col):
    grid_spec = pltpu.PrefetchScalarGridSpec(
        num_scalar_prefetch=2,
        grid=(NB,),
        in_specs=[
            pl.BlockSpec((B, H), lambda b, be, na: (b, 0)),
            pl.BlockSpec((1, D, F), lambda b, be, na: (be[b], 0, 0)),
            pl.BlockSpec((1, 1, F), lambda b, be, na: (be[b], 0, 0)),
            pl.BlockSpec((1, F, D), lambda b, be, na: (be[b], 0, 0)),
            pl.BlockSpec((1, 1, D), lambda b, be, na: (be[b], 0, 0)),
            pl.BlockSpec((B, 1), lambda b, be, na: (b, 0)),
        ],
        out_specs=pl.BlockSpec((B, H), lambda b, be, na: (b, 0)),
        scratch_shapes=[
            pltpu.VMEM((D, F), jnp.bfloat16),
            pltpu.VMEM((F, D), jnp.bfloat16),
        ],
    )
    return pl.pallas_call(
        _ffn_body,
        grid_spec=grid_spec,
        out_shape=jax.ShapeDtypeStruct((NPAD, H), jnp.int32),
        compiler_params=pltpu.CompilerParams(
            dimension_semantics=("arbitrary",)),
    )(block_expert, n_active, xs, W1, b1r, W2, b2r, gate_col)


CC = 32
TPW = T // NW
CCH = TPW // CC
CNS = 3
_HI = -65536


@functools.partial(
    pl.kernel,
    mesh=_SC_MESH,
    out_type=jax.ShapeDtypeStruct((T, H), jnp.int32),
    scratch_types=(
        [pltpu.VMEM((TPW,), jnp.int32), pltpu.VMEM((TPW,), jnp.int32)]
        + [pltpu.VMEM((CC, H), jnp.int32) for _ in range(2 * CNS)]
        + [pltpu.SemaphoreType.DMA for _ in range(2 * CNS)]
    ),
)
def _sc_pairsum(ys_hbm, pos0_hbm, pos1_hbm, out_hbm, p0_v, p1_v, *rest):
    r0s = rest[:CNS]
    r1s = rest[CNS:2 * CNS]
    dsems = rest[2 * CNS:3 * CNS]
    wsems = rest[3 * CNS:]
    wid = lax.axis_index("s") * 2 + lax.axis_index("c")
    base = wid * TPW
    pltpu.sync_copy(pos0_hbm.at[pl.ds(base, TPW)], p0_v)
    pltpu.sync_copy(pos1_hbm.at[pl.ds(base, TPW)], p1_v)
    dh = {}
    wh = {}

    def issue(c):
        k = c % CNS
        dh[c] = (
            pltpu.async_copy(
                ys_hbm.at[p0_v.at[pl.ds(c * CC, CC)]], r0s[k], dsems[k]),
            pltpu.async_copy(
                ys_hbm.at[p1_v.at[pl.ds(c * CC, CC)]], r1s[k], dsems[k]),
        )

    for c in range(min(CNS, CCH)):
        issue(c)
    for c in range(CCH):
        if 1 <= c and c + CNS - 1 < CCH:
            wh[c - 1].wait()
            issue(c + CNS - 1)
        for hnd in dh[c]:
            hnd.wait()
        k = c % CNS
        r0, r1 = r0s[k], r1s[k]

        def _rne16(u):
            return (u + 0x7FFF + ((u >> 16) & 1)) >> 16

        def _row(i, _):
            for j in range(H // 16):
                sl = pl.ds(j * 16, 16)
                v0 = r0[i, sl]
                v1 = r1[i, sl]
                lo = (lax.bitcast_convert_type(v0 << 16, jnp.float32)
                      + lax.bitcast_convert_type(v1 << 16, jnp.float32))
                hi = (lax.bitcast_convert_type(v0 & _HI, jnp.float32)
                      + lax.bitcast_convert_type(v1 & _HI, jnp.float32))
                ulo = lax.bitcast_convert_type(lo, jnp.uint32)
                uhi = lax.bitcast_convert_type(hi, jnp.uint32)
                packed = _rne16(ulo) | (_rne16(uhi) << 16)
                r0[i, sl] = lax.bitcast_convert_type(packed, jnp.int32)
            return 0

        lax.fori_loop(0, CC, _row, 0)
        wh[c] = pltpu.async_copy(
            r0, out_hbm.at[pl.ds(base + c * CC, CC)], wsems[k])
    for c in range(max(0, CCH - CNS), CCH):
        wh[c].wait()


def _finish_body(cheap_ref, rs_ref, out_ref):
    out_ref[...] = cheap_ref[...] + _unpack_cols(rs_ref[...])


def _finish(cheap, rsum):
    return pl.pallas_call(
        _finish_body,
        grid=(T // RB,),
        in_specs=[
            pl.BlockSpec((RB, D), lambda i: (i, 0)),
            pl.BlockSpec((RB, H), lambda i: (i, 0)),
        ],
        out_specs=pl.BlockSpec((RB, D), lambda i: (i, 0)),
        out_shape=jax.ShapeDtypeStruct((T, D), jnp.float32),
    )(cheap, rsum)


def kernel(x, wg, W1, b1, W2, b2, cw, cconst):
    xf = x.reshape(T, D)
    cwf = jnp.concatenate([cw[0], cw[1]], axis=1)
    cheap, gates, idx, xp = _router(xf, wg, cwf, cconst)

    pair_e = idx.reshape(-1)
    pair_g = gates.reshape(-1)
    pair_t = jnp.repeat(jnp.arange(T, dtype=jnp.int32), TOPK)
    is_ffn = pair_e < NF
    ec = jnp.where(is_ffn, pair_e, 0)
    onehot = (pair_e[:, None]
              == jnp.arange(NF, dtype=jnp.int32)[None, :]).astype(jnp.int32)
    csum = jnp.cumsum(onehot, axis=0)
    rank = jnp.take_along_axis(csum, ec[:, None], axis=1)[:, 0] - 1
    counts = csum[-1]
    padded = ((counts + B - 1) // B) * B
    po = jnp.concatenate(
        [jnp.zeros((1,), jnp.int32), jnp.cumsum(padded)]).astype(jnp.int32)
    dest = po[ec] + rank
    dest_s = jnp.where(is_ffn, dest, NPAD)
    tok_sorted = jnp.zeros((NPAD,), jnp.int32).at[dest_s].set(
        pair_t, mode="drop")
    gate_sorted = jnp.zeros((NPAD,), jnp.float32).at[dest_s].set(
        pair_g, mode="drop")
    pos = jnp.where(is_ffn, dest, ZROW).reshape(T, TOPK)
    n_active = (po[NF] // B).reshape(1).astype(jnp.int32)
    bstart = jnp.arange(NB, dtype=jnp.int32) * B
    block_expert = jnp.minimum(
        jnp.sum((bstart[:, None] >= po[None, 1:NF + 1]).astype(jnp.int32),
                axis=1),
        NF - 1).astype(jnp.int32)

    xs = _sc_gather(xp, tok_sorted)

    b1r = b1.reshape(NF, 1, F)
    b2r = b2.reshape(NF, 1, D)
    ys = _ffn(block_expert, n_active, xs, W1, b1r, W2, b2r,
              gate_sorted[:, None])

    pos0 = pos[:, 0] + 0
    pos1 = pos[:, 1] + 0
    rsum = _sc_pairsum(ys, pos0, pos1)
    out = _finish(cheap, rsum)
    return out.reshape(x.shape)
